# Initial kernel scaffold; baseline (speedup 1.0000x reference)
import jax
import jax.numpy as jnp
from jax.experimental import pallas as pl


def kernel(x, hardness):
    raise NotImplementedError("write your pallas kernel here")


# speedup vs baseline: 16.3584x; 16.3584x over previous
import jax
import jax.numpy as jnp
from jax.experimental import pallas as pl
from jax.experimental.pallas import tpu as pltpu

K_ACTIVE = 512
INT_MIN = -2147483648
LOW31 = 2147483647


def _kwta_tc_kernel(hard_ref, x_ref, o_ref):
    x = x_ref[...]
    b = jax.lax.bitcast_convert_type(x, jnp.int32)
    e = b ^ (jax.lax.shift_right_arithmetic(b, 31) & LOW31)
    eu = e ^ INT_MIN

    rows = x.shape[0]
    prefix0 = jnp.zeros((rows, 1), jnp.int32)
    k0 = jnp.full((rows, 1), K_ACTIVE, jnp.int32)

    def body(t, carry):
        prefix, k = carry
        i = 31 - t
        bit = jnp.int32(1) << i
        mask_high = jnp.int32(-1) << i
        test = prefix | bit
        c1 = jnp.sum(((eu & mask_high) == test).astype(jnp.int32), axis=1,
                     keepdims=True)
        take_hi = k < c1
        prefix = jnp.where(take_hi, test, prefix)
        k = jnp.where(take_hi, k, k - c1)
        return prefix, k

    prefix, _ = jax.lax.fori_loop(0, 32, body, (prefix0, k0))

    e1 = prefix ^ INT_MIN
    b1 = e1 ^ (jax.lax.shift_right_arithmetic(e1, 31) & LOW31)
    v1 = jax.lax.bitcast_convert_type(b1, jnp.float32)

    c_ge = jnp.sum((x >= v1).astype(jnp.int32), axis=1, keepdims=True)
    neg_inf = jnp.float32(-jnp.inf)
    below_max = jnp.max(jnp.where(x < v1, x, neg_inf), axis=1, keepdims=True)
    v2 = jnp.where(c_ge >= K_ACTIVE + 2, v1, below_max)

    thr = (v1 + v2) * 0.5
    hard = hard_ref[0]
    o_ref[...] = jax.nn.sigmoid(hard * (x - thr))


def kernel(x, hardness):
    hard = jnp.reshape(hardness, (1,)).astype(jnp.float32)
    return pl.pallas_call(
        _kwta_tc_kernel,
        out_shape=jax.ShapeDtypeStruct(x.shape, jnp.float32),
        in_specs=[
            pl.BlockSpec(memory_space=pltpu.SMEM),
            pl.BlockSpec(memory_space=pltpu.VMEM),
        ],
        out_specs=pl.BlockSpec(memory_space=pltpu.VMEM),
    )(hard, x)
